# baseline (speedup 1.0000x reference)
import jax
import jax.numpy as jnp
import numpy as np
from jax import lax
from jax.experimental import pallas as pl
from jax.experimental.pallas import tpu as pltpu
from jax.experimental.pallas import tpu_sc as plsc

N_NODES = 40962
FEAT = 128
HALF = FEAT // 2
NUM_NEW = 3 * N_NODES - 6
N_IDX = 2 * NUM_NEW
N_OUT = N_NODES + NUM_NEW

NC, NS = 2, 16
NW = NC * NS

N_GW = 24
N_CW = NW - N_GW
NEW_PER_GW = NUM_NEW // N_GW
IDX_PER_GW = 2 * NEW_PER_GW

GROWS = 256
NGRP = NEW_PER_GW // GROWS
GB = GROWS // 128

TOP_PER_CW = N_NODES // N_CW
TOP_REM = N_NODES - TOP_PER_CW * N_CW
N_CCH = TOP_PER_CW // GROWS

_ZBLK = 4096


N2 = N_NODES // 2


def _tc_body(x2_ref, a_ref, z_ref):
    xv = x2_ref[...]
    kw = dict(precision=lax.Precision.HIGHEST,
              preferred_element_type=jnp.float32)
    ze = jnp.dot(xv[:, :FEAT], a_ref[...], **kw)
    zo = jnp.dot(xv[:, FEAT:], a_ref[...], **kw)
    z_ref[...] = jnp.concatenate([ze, zo], axis=1)


def _make_avg_matrix():
    a = np.zeros((FEAT, HALF), np.float32)
    for f in range(HALF):
        a[2 * f, f] = 0.5
        a[2 * f + 1, f] = 0.5
    return jnp.asarray(a)


def _compute_z(x):
    n_blk = (N2 + _ZBLK - 1) // _ZBLK
    z128 = pl.pallas_call(
        _tc_body,
        grid=(n_blk,),
        in_specs=[
            pl.BlockSpec((_ZBLK, 2 * FEAT), lambda i: (i, 0)),
            pl.BlockSpec((FEAT, HALF), lambda i: (0, 0)),
        ],
        out_specs=pl.BlockSpec((_ZBLK, FEAT), lambda i: (i, 0)),
        out_shape=jax.ShapeDtypeStruct((N2, FEAT), jnp.float32),
    )(x.reshape(N2, 2 * FEAT), _make_avg_matrix())
    return z128.reshape(N_NODES, HALF)


def _sc_body(x_hbm, z_hbm, idx_hbm, out_hbm,
             idx_v, idx_e, idx_o, buf_ea, buf_oa, buf_eb, buf_ob,
             sem_ia, sem_ib, sem_oa, sem_ob):
    wid = lax.axis_index("s") * NC + lax.axis_index("c")

    @pl.when(wid < N_GW)
    def _gather():
        pltpu.sync_copy(idx_hbm.at[pl.ds(wid * IDX_PER_GW, IDX_PER_GW)],
                        idx_v)

        lane = lax.iota(jnp.int32, 16)

        def deint(k, carry):
            base = 32 * k
            ev = plsc.load_gather(idx_v, [base + 2 * lane])
            od = plsc.load_gather(idx_v, [base + 2 * lane + 1])
            idx_e[pl.ds(16 * k, 16)] = ev
            idx_o[pl.ds(16 * k, 16)] = od
            return carry

        lax.fori_loop(0, IDX_PER_GW // 32, deint, 0)

        def run_group(i, g, b_e, b_o, sem_i, sem_o):
            @pl.when(i > 0)
            def _():
                pltpu.make_async_copy(z_hbm.at[pl.ds(0, GROWS)], b_e,
                                      sem_o).wait()
                pltpu.make_async_copy(z_hbm.at[pl.ds(0, GROWS)], b_o,
                                      sem_o).wait()
            descs = []
            for b in range(GB):
                sl = pl.ds(g * GROWS + 128 * b, 128)
                dsl = pl.ds(128 * b, 128)
                descs.append(pltpu.async_copy(
                    z_hbm.at[idx_e.at[sl]], b_e.at[dsl],
                    sem_i))
                descs.append(pltpu.async_copy(
                    z_hbm.at[idx_o.at[sl]], b_o.at[dsl],
                    sem_i))
            for d in descs:
                d.wait()
            row = N_NODES + wid * NEW_PER_GW + g * GROWS
            pltpu.async_copy(
                b_e, out_hbm.at[pl.ds(row, GROWS), pl.ds(0, HALF)], sem_o)
            pltpu.async_copy(
                b_o, out_hbm.at[pl.ds(row, GROWS), pl.ds(HALF, HALF)], sem_o)

        def body(i, carry):
            run_group(i, 2 * i, buf_ea, buf_oa, sem_ia, sem_oa)
            run_group(i, 2 * i + 1, buf_eb, buf_ob, sem_ib, sem_ob)
            return carry

        lax.fori_loop(0, NGRP // 2, body, 0)

        for b_x, sem in ((buf_ea, sem_oa), (buf_oa, sem_oa),
                         (buf_eb, sem_ob), (buf_ob, sem_ob)):
            pltpu.make_async_copy(z_hbm.at[pl.ds(0, GROWS)], b_x, sem).wait()

    @pl.when(wid >= N_GW)
    def _copy():
        cw = wid - N_GW
        for c in range(N_CCH):
            b_e, b_o, sem_i, sem_o = ((buf_ea, buf_oa, sem_ia, sem_oa)
                                      if c % 2 == 0 else
                                      (buf_eb, buf_ob, sem_ib, sem_ob))
            base = cw * TOP_PER_CW + c * GROWS
            rows = pl.ds(base, GROWS)
            if c >= 2:
                pltpu.make_async_copy(z_hbm.at[pl.ds(0, GROWS)], b_e,
                                      sem_o).wait()
                pltpu.make_async_copy(z_hbm.at[pl.ds(0, GROWS)], b_o,
                                      sem_o).wait()
            dl = pltpu.async_copy(x_hbm.at[rows, pl.ds(0, HALF)], b_e, sem_i)
            dr = pltpu.async_copy(x_hbm.at[rows, pl.ds(HALF, HALF)], b_o,
                                  sem_i)
            dl.wait()
            dr.wait()
            pltpu.async_copy(b_e, out_hbm.at[rows, pl.ds(0, HALF)], sem_o)
            pltpu.async_copy(b_o, out_hbm.at[rows, pl.ds(HALF, HALF)], sem_o)
        for b_x, sem in ((buf_ea, sem_oa), (buf_oa, sem_oa),
                         (buf_eb, sem_ob), (buf_ob, sem_ob)):
            pltpu.make_async_copy(z_hbm.at[pl.ds(0, GROWS)], b_x, sem).wait()

        @pl.when(wid == NW - 1)
        def _():
            tail = pl.ds(TOP_PER_CW * N_CW, TOP_REM)
            rsl = pl.ds(0, TOP_REM)
            pltpu.sync_copy(x_hbm.at[tail, pl.ds(0, HALF)],
                            buf_ea.at[rsl])
            pltpu.sync_copy(buf_ea.at[rsl],
                            out_hbm.at[tail, pl.ds(0, HALF)])
            pltpu.sync_copy(x_hbm.at[tail, pl.ds(HALF, HALF)],
                            buf_oa.at[rsl])
            pltpu.sync_copy(buf_oa.at[rsl],
                            out_hbm.at[tail, pl.ds(HALF, HALF)])


@jax.jit
def _run(x, idx32):
    z = _compute_z(x)
    mesh = plsc.VectorSubcoreMesh(core_axis_name="c", subcore_axis_name="s",
                                  num_cores=NC, num_subcores=NS)
    return pl.kernel(
        _sc_body,
        out_type=jax.ShapeDtypeStruct((N_OUT, FEAT), jnp.float32),
        mesh=mesh,
        compiler_params=pltpu.CompilerParams(use_tc_tiling_on_sc=False,
                                             needs_layout_passes=False),
        scratch_types=[
            pltpu.VMEM((IDX_PER_GW,), jnp.int32),
            pltpu.VMEM((NEW_PER_GW,), jnp.int32),
            pltpu.VMEM((NEW_PER_GW,), jnp.int32),
            pltpu.VMEM((GROWS, HALF), jnp.float32),
            pltpu.VMEM((GROWS, HALF), jnp.float32),
            pltpu.VMEM((GROWS, HALF), jnp.float32),
            pltpu.VMEM((GROWS, HALF), jnp.float32),
            pltpu.SemaphoreType.DMA,
            pltpu.SemaphoreType.DMA,
            pltpu.SemaphoreType.DMA,
            pltpu.SemaphoreType.DMA,
        ],
    )(x, z, idx32)


def kernel(x, upsample_neighs_order):
    return _run(x, upsample_neighs_order.astype(jnp.int32))


# speedup vs baseline: 1.2113x; 1.2113x over previous
import jax
import jax.numpy as jnp
import numpy as np
from jax import lax
from jax.experimental import pallas as pl
from jax.experimental.pallas import tpu as pltpu
from jax.experimental.pallas import tpu_sc as plsc

N_NODES = 40962
FEAT = 128
HALF = FEAT // 2
NUM_NEW = 3 * N_NODES - 6
N_IDX = 2 * NUM_NEW
N_OUT = N_NODES + NUM_NEW

NC, NS = 2, 16
NW = NC * NS

N_GW = 24
N_CW = NW - N_GW
NEW_PER_GW = NUM_NEW // N_GW
IDX_PER_GW = 2 * NEW_PER_GW

GROWS = 256
NGRP = NEW_PER_GW // GROWS
GB = GROWS // 128

TOP_PER_CW = N_NODES // N_CW
TOP_REM = N_NODES - TOP_PER_CW * N_CW
N_CCH = TOP_PER_CW // GROWS

_ZBLK = 4096


ZC = 6 * _ZBLK


def _tc_body(xa_ref, xb_ref, a_ref, z_ref):
    kw = dict(precision=lax.Precision.HIGHEST,
              preferred_element_type=jnp.float32)
    za = jnp.dot(xa_ref[...], a_ref[...], **kw)
    zb = jnp.dot(xb_ref[...], a_ref[...], **kw)
    z_ref[...] = jnp.concatenate([za, zb], axis=1)


def _make_avg_matrix():
    a = np.zeros((FEAT, HALF), np.float32)
    for f in range(HALF):
        a[2 * f, f] = 0.5
        a[2 * f + 1, f] = 0.5
    return jnp.asarray(a)


def _compute_z(x):
    n_blk = ZC // _ZBLK
    z2 = pl.pallas_call(
        _tc_body,
        grid=(n_blk,),
        in_specs=[
            pl.BlockSpec((_ZBLK, FEAT), lambda i: (i, 0)),
            pl.BlockSpec((_ZBLK, FEAT),
                         lambda i: (jnp.minimum(i + n_blk, 2 * n_blk - 2),
                                    0)),
            pl.BlockSpec((FEAT, HALF), lambda i: (0, 0)),
        ],
        out_specs=pl.BlockSpec((_ZBLK, FEAT), lambda i: (i, 0)),
        out_shape=jax.ShapeDtypeStruct((ZC, FEAT), jnp.float32),
    )(x, x, _make_avg_matrix())
    return z2.reshape(2 * ZC, HALF)


def _sc_body(x_hbm, z_hbm, idx_hbm, out_hbm,
             idx_v, idx_e, idx_o, buf_ea, buf_oa, buf_eb, buf_ob,
             sem_ia, sem_ib, sem_oa, sem_ob):
    wid = lax.axis_index("s") * NC + lax.axis_index("c")

    @pl.when(wid < N_GW)
    def _gather():
        pltpu.sync_copy(idx_hbm.at[pl.ds(wid * IDX_PER_GW, IDX_PER_GW)],
                        idx_v)

        lane = lax.iota(jnp.int32, 16)

        def tr(v):
            return jnp.where(v < ZC, 2 * v, 2 * v - (2 * ZC - 1))

        def deint(k, carry):
            base = 32 * k
            ev = plsc.load_gather(idx_v, [base + 2 * lane])
            od = plsc.load_gather(idx_v, [base + 2 * lane + 1])
            idx_e[pl.ds(16 * k, 16)] = tr(ev)
            idx_o[pl.ds(16 * k, 16)] = tr(od)
            return carry

        lax.fori_loop(0, IDX_PER_GW // 32, deint, 0)

        def run_group(i, g, b_e, b_o, sem_i, sem_o):
            @pl.when(i > 0)
            def _():
                pltpu.make_async_copy(z_hbm.at[pl.ds(0, GROWS)], b_e,
                                      sem_o).wait()
                pltpu.make_async_copy(z_hbm.at[pl.ds(0, GROWS)], b_o,
                                      sem_o).wait()
            descs = []
            for b in range(GB):
                sl = pl.ds(g * GROWS + 128 * b, 128)
                dsl = pl.ds(128 * b, 128)
                descs.append(pltpu.async_copy(
                    z_hbm.at[idx_e.at[sl]], b_e.at[dsl],
                    sem_i))
                descs.append(pltpu.async_copy(
                    z_hbm.at[idx_o.at[sl]], b_o.at[dsl],
                    sem_i))
            for d in descs:
                d.wait()
            row = N_NODES + wid * NEW_PER_GW + g * GROWS
            pltpu.async_copy(
                b_e, out_hbm.at[pl.ds(row, GROWS), pl.ds(0, HALF)], sem_o)
            pltpu.async_copy(
                b_o, out_hbm.at[pl.ds(row, GROWS), pl.ds(HALF, HALF)], sem_o)

        def body(i, carry):
            run_group(i, 2 * i, buf_ea, buf_oa, sem_ia, sem_oa)
            run_group(i, 2 * i + 1, buf_eb, buf_ob, sem_ib, sem_ob)
            return carry

        lax.fori_loop(0, NGRP // 2, body, 0)

        for b_x, sem in ((buf_ea, sem_oa), (buf_oa, sem_oa),
                         (buf_eb, sem_ob), (buf_ob, sem_ob)):
            pltpu.make_async_copy(z_hbm.at[pl.ds(0, GROWS)], b_x, sem).wait()

    @pl.when(wid >= N_GW)
    def _copy():
        cw = wid - N_GW
        for c in range(N_CCH):
            b_e, b_o, sem_i, sem_o = ((buf_ea, buf_oa, sem_ia, sem_oa)
                                      if c % 2 == 0 else
                                      (buf_eb, buf_ob, sem_ib, sem_ob))
            base = cw * TOP_PER_CW + c * GROWS
            rows = pl.ds(base, GROWS)
            if c >= 2:
                pltpu.make_async_copy(z_hbm.at[pl.ds(0, GROWS)], b_e,
                                      sem_o).wait()
                pltpu.make_async_copy(z_hbm.at[pl.ds(0, GROWS)], b_o,
                                      sem_o).wait()
            dl = pltpu.async_copy(x_hbm.at[rows, pl.ds(0, HALF)], b_e, sem_i)
            dr = pltpu.async_copy(x_hbm.at[rows, pl.ds(HALF, HALF)], b_o,
                                  sem_i)
            dl.wait()
            dr.wait()
            pltpu.async_copy(b_e, out_hbm.at[rows, pl.ds(0, HALF)], sem_o)
            pltpu.async_copy(b_o, out_hbm.at[rows, pl.ds(HALF, HALF)], sem_o)
        for b_x, sem in ((buf_ea, sem_oa), (buf_oa, sem_oa),
                         (buf_eb, sem_ob), (buf_ob, sem_ob)):
            pltpu.make_async_copy(z_hbm.at[pl.ds(0, GROWS)], b_x, sem).wait()

        @pl.when(wid == NW - 1)
        def _():
            tail = pl.ds(TOP_PER_CW * N_CW, TOP_REM)
            rsl = pl.ds(0, TOP_REM)
            pltpu.sync_copy(x_hbm.at[tail, pl.ds(0, HALF)],
                            buf_ea.at[rsl])
            pltpu.sync_copy(buf_ea.at[rsl],
                            out_hbm.at[tail, pl.ds(0, HALF)])
            pltpu.sync_copy(x_hbm.at[tail, pl.ds(HALF, HALF)],
                            buf_oa.at[rsl])
            pltpu.sync_copy(buf_oa.at[rsl],
                            out_hbm.at[tail, pl.ds(HALF, HALF)])


@jax.jit
def _run(x, idx32):
    z = _compute_z(x)
    mesh = plsc.VectorSubcoreMesh(core_axis_name="c", subcore_axis_name="s",
                                  num_cores=NC, num_subcores=NS)
    return pl.kernel(
        _sc_body,
        out_type=jax.ShapeDtypeStruct((N_OUT, FEAT), jnp.float32),
        mesh=mesh,
        compiler_params=pltpu.CompilerParams(use_tc_tiling_on_sc=False,
                                             needs_layout_passes=False),
        scratch_types=[
            pltpu.VMEM((IDX_PER_GW,), jnp.int32),
            pltpu.VMEM((NEW_PER_GW,), jnp.int32),
            pltpu.VMEM((NEW_PER_GW,), jnp.int32),
            pltpu.VMEM((GROWS, HALF), jnp.float32),
            pltpu.VMEM((GROWS, HALF), jnp.float32),
            pltpu.VMEM((GROWS, HALF), jnp.float32),
            pltpu.VMEM((GROWS, HALF), jnp.float32),
            pltpu.SemaphoreType.DMA,
            pltpu.SemaphoreType.DMA,
            pltpu.SemaphoreType.DMA,
            pltpu.SemaphoreType.DMA,
        ],
    )(x, z, idx32)


def kernel(x, upsample_neighs_order):
    return _run(x, upsample_neighs_order.astype(jnp.int32))
